# baseline (speedup 1.0000x reference)
import functools

import jax
import jax.numpy as jnp
from jax import lax
from jax.experimental import pallas as pl
from jax.experimental.pallas import tpu as pltpu
from jax.experimental.pallas import tpu_sc as plsc

_LANES = 16
_NC = 2
_NS = 16
_NW = _NC * _NS


@functools.lru_cache(maxsize=None)
def _build_gather_kernel(B, T):
    bpw = B // _NW
    assert bpw * _NW == B and bpw % _LANES == 0

    scratch = [
        pltpu.VMEM((bpw,), jnp.int32),
        pltpu.VMEM((T,), jnp.float32),
        pltpu.VMEM((T,), jnp.float32),
        pltpu.VMEM((bpw,), jnp.float32),
        pltpu.VMEM((bpw,), jnp.float32),
    ]
    mesh = plsc.VectorSubcoreMesh(core_axis_name="c", subcore_axis_name="s")

    @functools.partial(
        pl.kernel,
        out_type=(
            jax.ShapeDtypeStruct((B,), jnp.float32),
            jax.ShapeDtypeStruct((B,), jnp.float32),
        ),
        mesh=mesh,
        scratch_types=scratch,
        compiler_params=pltpu.CompilerParams(needs_layout_passes=False),
    )
    def k(t_hbm, a1_hbm, a2_hbm, c1_hbm, c2_hbm, t_v, tab1_v, tab2_v, c1_v, c2_v):
        wid = lax.axis_index("s") * _NC + lax.axis_index("c")
        base = wid * bpw
        pltpu.sync_copy(t_hbm.at[pl.ds(base, bpw)], t_v)
        pltpu.sync_copy(a1_hbm, tab1_v)
        pltpu.sync_copy(a2_hbm, tab2_v)
        for i in range(bpw // _LANES):
            sl = pl.ds(i * _LANES, _LANES)
            idx = t_v[sl]
            c1_v[sl] = plsc.load_gather(tab1_v, [idx])
            c2_v[sl] = plsc.load_gather(tab2_v, [idx])
        pltpu.sync_copy(c1_v, c1_hbm.at[pl.ds(base, bpw)])
        pltpu.sync_copy(c2_v, c2_hbm.at[pl.ds(base, bpw)])

    return k


def _combine_body(c1_ref, c2_ref, x_ref, n_ref, o_ref):
    G = c1_ref.shape[1]
    ii = lax.broadcasted_iota(jnp.int32, (128, 128), 0)
    jj = lax.broadcasted_iota(jnp.int32, (128, 128), 1)
    eye = (ii == jj).astype(jnp.float32)
    ones = jnp.ones((128, 128), jnp.float32)
    for k in range(G):
        c1row = c1_ref[0, pl.ds(k, 1), :]
        c2row = c2_ref[0, pl.ds(k, 1), :]
        cb1 = jnp.dot(eye * c1row, ones, precision=lax.Precision.HIGHEST,
                      preferred_element_type=jnp.float32)
        cb2 = jnp.dot(eye * c2row, ones, precision=lax.Precision.HIGHEST,
                      preferred_element_type=jnp.float32)
        sub = pl.ds(k * 128, 128)
        o_ref[sub, :] = cb1 * x_ref[sub, :] + cb2 * n_ref[sub, :]


@functools.lru_cache(maxsize=None)
def _build_combine_kernel(B, D):
    BK = 512
    G = BK // 128
    assert B % BK == 0 and D == 128
    return pl.pallas_call(
        _combine_body,
        grid=(B // BK,),
        in_specs=[
            pl.BlockSpec((1, G, 128), lambda i: (i, 0, 0)),
            pl.BlockSpec((1, G, 128), lambda i: (i, 0, 0)),
            pl.BlockSpec((BK, D), lambda i: (i, 0)),
            pl.BlockSpec((BK, D), lambda i: (i, 0)),
        ],
        out_specs=pl.BlockSpec((BK, D), lambda i: (i, 0)),
        out_shape=jax.ShapeDtypeStruct((B, D), jnp.float32),
    )


def kernel(x_start, t, noise, sqrt_alphas_cumprod, sqrt_one_minus_alphas_cumprod):
    B, D = x_start.shape
    T = sqrt_alphas_cumprod.shape[0]
    c1, c2 = _build_gather_kernel(B, T)(
        t, sqrt_alphas_cumprod, sqrt_one_minus_alphas_cumprod
    )
    c1 = c1.reshape(B // 512, 4, 128)
    c2 = c2.reshape(B // 512, 4, 128)
    return _build_combine_kernel(B, D)(c1, c2, x_start, noise)


# speedup vs baseline: 1.5001x; 1.5001x over previous
import functools

import jax
import jax.numpy as jnp
from jax import lax
from jax.experimental import pallas as pl
from jax.experimental.pallas import tpu as pltpu
from jax.experimental.pallas import tpu_sc as plsc

_LANES = 16
_NC = 2
_NS = 16
_NW = _NC * _NS


@functools.lru_cache(maxsize=None)
def _build_sc_kernel(B, D, T):
    bpw = B // _NW
    CH = 128
    NBUF = 2
    NCHUNK = bpw // CH
    assert bpw * _NW == B and NCHUNK * CH == bpw and D % _LANES == 0

    scratch = [
        pltpu.VMEM((bpw,), jnp.int32),
        pltpu.VMEM((T,), jnp.float32),
        pltpu.VMEM((T,), jnp.float32),
        pltpu.VMEM((bpw,), jnp.float32),
        pltpu.VMEM((bpw,), jnp.float32),
    ]
    scratch += [pltpu.VMEM((CH, D), jnp.float32) for _ in range(3 * NBUF)]
    scratch += [pltpu.SemaphoreType.DMA] * (3 * NBUF)

    mesh = plsc.VectorSubcoreMesh(core_axis_name="c", subcore_axis_name="s")

    @functools.partial(
        pl.kernel,
        out_type=jax.ShapeDtypeStruct((B, D), jnp.float32),
        mesh=mesh,
        scratch_types=scratch,
        compiler_params=pltpu.CompilerParams(needs_layout_passes=False),
    )
    def k(x_hbm, t_hbm, n_hbm, a1_hbm, a2_hbm, out_hbm, *rest):
        t_v, tab1_v, tab2_v, c1_v, c2_v = rest[:5]
        xbufs = rest[5:5 + NBUF]
        nbufs = rest[5 + NBUF:5 + 2 * NBUF]
        obufs = rest[5 + 2 * NBUF:5 + 3 * NBUF]
        sems = rest[5 + 3 * NBUF:]
        sx, sn, so = sems[:NBUF], sems[NBUF:2 * NBUF], sems[2 * NBUF:]

        wid = lax.axis_index("s") * _NC + lax.axis_index("c")
        base = wid * bpw

        def start_in(g):
            b = g % NBUF
            r0 = base + g * CH
            pltpu.async_copy(x_hbm.at[pl.ds(r0, CH), :], xbufs[b], sx[b])
            pltpu.async_copy(n_hbm.at[pl.ds(r0, CH), :], nbufs[b], sn[b])

        def wait_in(g):
            b = g % NBUF
            r0 = base + g * CH
            pltpu.make_async_copy(x_hbm.at[pl.ds(r0, CH), :], xbufs[b], sx[b]).wait()
            pltpu.make_async_copy(n_hbm.at[pl.ds(r0, CH), :], nbufs[b], sn[b]).wait()

        def start_out(g):
            b = g % NBUF
            r0 = base + g * CH
            pltpu.async_copy(obufs[b], out_hbm.at[pl.ds(r0, CH), :], so[b])

        def wait_out(g):
            b = g % NBUF
            r0 = base + g * CH
            pltpu.make_async_copy(obufs[b], out_hbm.at[pl.ds(r0, CH), :], so[b]).wait()

        for g in range(min(NBUF, NCHUNK)):
            start_in(g)

        pltpu.sync_copy(t_hbm.at[pl.ds(base, bpw)], t_v)
        pltpu.sync_copy(a1_hbm, tab1_v)
        pltpu.sync_copy(a2_hbm, tab2_v)
        for i in range(bpw // _LANES):
            sl = pl.ds(i * _LANES, _LANES)
            idx = t_v[sl]
            c1_v[sl] = plsc.load_gather(tab1_v, [idx])
            c2_v[sl] = plsc.load_gather(tab2_v, [idx])

        for g in range(NCHUNK):
            b = g % NBUF
            wait_in(g)
            if g >= NBUF:
                wait_out(g - NBUF)

            w0 = g * CH

            @plsc.parallel_loop(0, CH, unroll=4)
            def row(r):
                idx = jnp.full((_LANES,), w0 + r, dtype=jnp.int32)
                c1 = plsc.load_gather(c1_v, [idx])
                c2 = plsc.load_gather(c2_v, [idx])
                for j in range(D // _LANES):
                    slj = pl.ds(j * _LANES, _LANES)
                    xv = xbufs[b][r, slj]
                    nv = nbufs[b][r, slj]
                    obufs[b][r, slj] = c1 * xv + c2 * nv

            start_out(g)
            if g + NBUF < NCHUNK:
                start_in(g + NBUF)

        for g in range(max(0, NCHUNK - NBUF), NCHUNK):
            wait_out(g)

    return k


def kernel(x_start, t, noise, sqrt_alphas_cumprod, sqrt_one_minus_alphas_cumprod):
    B, D = x_start.shape
    T = sqrt_alphas_cumprod.shape[0]
    k = _build_sc_kernel(B, D, T)
    return k(x_start, t, noise, sqrt_alphas_cumprod, sqrt_one_minus_alphas_cumprod)
